# baseline (speedup 1.0000x reference)
import jax
import jax.numpy as jnp
from jax.experimental import pallas as pl
from jax.experimental.pallas import tpu as pltpu

N = 100000
C = 128
NUM_REL = 16
ZROWS = 1000
NZ = N // ZROWS


def _kgat_kernel(x_hbm, t_hbm, w_hbm, rel_hbm, att_hbm, out_ref,
                 zbuf, zbuf2, zbuf3, zbuf4, live, xb, tb, wb, relb, attb,
                 zsems, insems):
    zbufs = (zbuf, zbuf2, zbuf3, zbuf4)
    for zb in zbufs:
        zb[...] = jnp.zeros_like(zb)
    copies = []
    for k in range(1, NZ):
        copies.append(pltpu.make_async_copy(
            zbufs[k % 4], out_ref.at[pl.ds(k * ZROWS, ZROWS), :], zsems.at[k]))
    copies.append(pltpu.make_async_copy(
        zbuf.at[pl.ds(C, ZROWS - C), :],
        out_ref.at[pl.ds(C, ZROWS - C), :], zsems.at[0]))
    for cpy in copies:
        cpy.start()

    fetches = [
        pltpu.make_async_copy(x_hbm.at[pl.ds(0, C), :], xb, insems.at[0]),
        pltpu.make_async_copy(t_hbm, tb, insems.at[1]),
        pltpu.make_async_copy(w_hbm, wb, insems.at[2]),
        pltpu.make_async_copy(rel_hbm, relb, insems.at[3]),
        pltpu.make_async_copy(att_hbm, attb, insems.at[4]),
    ]
    for f in fetches:
        f.start()
    for f in fetches:
        f.wait()

    hi = jax.lax.Precision.HIGHEST
    lo = jax.lax.Precision.DEFAULT
    h = jax.lax.dot_general(
        xb[...], wb[...], (((1,), (1,)), ((), ())),
        preferred_element_type=jnp.float32, precision=lo)
    hb = h.astype(jnp.bfloat16).astype(jnp.float32)
    rb = relb[...].astype(jnp.bfloat16).astype(jnp.float32)
    t = tb[...]
    acc = jnp.zeros((C, C), jnp.float32)
    for r in range(NUM_REL):
        mask = (t == r).astype(jnp.float32)
        g = jax.lax.dot_general(
            mask, hb, (((0,), (0,)), ((), ())),
            preferred_element_type=jnp.float32, precision=hi)
        acc = acc + jax.lax.dot_general(
            g, rb[r * C:(r + 1) * C, :], (((1,), (0,)), ((), ())),
            preferred_element_type=jnp.float32, precision=hi)
    att = attb[0:1, :].astype(jnp.bfloat16).astype(jnp.float32)
    accb = acc.astype(jnp.bfloat16).astype(jnp.float32)
    s = jnp.sum(accb * att, axis=1, keepdims=True)
    m = jnp.maximum(jnp.max(s), 0.0)
    denom = jnp.sum(jnp.exp(s - m)) + (N - C) * jnp.exp(-m)
    wgt = jnp.exp(s - m) / denom
    live[...] = jnp.maximum(acc * wgt, 0.0)
    live_copy = pltpu.make_async_copy(
        live, out_ref.at[pl.ds(0, C), :], insems.at[5])
    live_copy.start()
    copies.append(live_copy)
    for cpy in copies:
        cpy.wait()


def kernel(x, edge_type, W, nodevec1, nodevec2, rel_table, attention):
    del nodevec1, nodevec2
    t = edge_type.reshape(C, C)
    rel = rel_table.reshape(NUM_REL * C, C)
    att = jnp.broadcast_to(attention.reshape(1, C), (8, C))
    any_spec = pl.BlockSpec(memory_space=pl.ANY)
    return pl.pallas_call(
        _kgat_kernel,
        grid=(1,),
        in_specs=[any_spec] * 5,
        out_specs=any_spec,
        out_shape=jax.ShapeDtypeStruct((N, C), jnp.float32),
        scratch_shapes=[
            pltpu.VMEM((ZROWS, C), jnp.float32),
            pltpu.VMEM((ZROWS, C), jnp.float32),
            pltpu.VMEM((ZROWS, C), jnp.float32),
            pltpu.VMEM((ZROWS, C), jnp.float32),
            pltpu.VMEM((C, C), jnp.float32),
            pltpu.VMEM((C, C), jnp.float32),
            pltpu.VMEM((C, C), jnp.int32),
            pltpu.VMEM((C, C), jnp.float32),
            pltpu.VMEM((NUM_REL * C, C), jnp.float32),
            pltpu.VMEM((8, C), jnp.float32),
            pltpu.SemaphoreType.DMA((NZ,)),
            pltpu.SemaphoreType.DMA((6,)),
        ],
    )(x, t, W, rel, att)


# speedup vs baseline: 1.0222x; 1.0222x over previous
import jax
import jax.numpy as jnp
from jax.experimental import pallas as pl
from jax.experimental.pallas import tpu as pltpu

N = 100000
C = 128
NUM_REL = 16
ZROWS = 4000
NZ = N // ZROWS


def _kgat_kernel(x_hbm, t_hbm, w_hbm, rel_hbm, att_hbm, out_ref,
                 zbuf, zbuf2, zbuf3, zbuf4, live, xb, tb, wb, relb, attb,
                 zsems, insems):
    zbufs = (zbuf, zbuf2, zbuf3, zbuf4)
    for zb in zbufs:
        zb[...] = jnp.zeros_like(zb)
    copies = []
    for k in range(1, NZ):
        copies.append(pltpu.make_async_copy(
            zbufs[k % 4], out_ref.at[pl.ds(k * ZROWS, ZROWS), :], zsems.at[k]))
    copies.append(pltpu.make_async_copy(
        zbuf.at[pl.ds(C, ZROWS - C), :],
        out_ref.at[pl.ds(C, ZROWS - C), :], zsems.at[0]))
    for cpy in copies:
        cpy.start()

    fetches = [
        pltpu.make_async_copy(x_hbm.at[pl.ds(0, C), :], xb, insems.at[0]),
        pltpu.make_async_copy(t_hbm, tb, insems.at[1]),
        pltpu.make_async_copy(w_hbm, wb, insems.at[2]),
        pltpu.make_async_copy(rel_hbm, relb, insems.at[3]),
        pltpu.make_async_copy(att_hbm, attb, insems.at[4]),
    ]
    for f in fetches:
        f.start()
    for f in fetches:
        f.wait()

    hi = jax.lax.Precision.HIGHEST
    lo = jax.lax.Precision.DEFAULT
    h = jax.lax.dot_general(
        xb[...], wb[...], (((1,), (1,)), ((), ())),
        preferred_element_type=jnp.float32, precision=lo)
    hb = h.astype(jnp.bfloat16).astype(jnp.float32)
    rb = relb[...].astype(jnp.bfloat16).astype(jnp.float32)
    t = tb[...]
    acc = jnp.zeros((C, C), jnp.float32)
    for r in range(NUM_REL):
        mask = (t == r).astype(jnp.float32)
        g = jax.lax.dot_general(
            mask, hb, (((0,), (0,)), ((), ())),
            preferred_element_type=jnp.float32, precision=hi)
        acc = acc + jax.lax.dot_general(
            g, rb[r * C:(r + 1) * C, :], (((1,), (0,)), ((), ())),
            preferred_element_type=jnp.float32, precision=hi)
    att = attb[0:1, :].astype(jnp.bfloat16).astype(jnp.float32)
    accb = acc.astype(jnp.bfloat16).astype(jnp.float32)
    s = jnp.sum(accb * att, axis=1, keepdims=True)
    m = jnp.maximum(jnp.max(s), 0.0)
    denom = jnp.sum(jnp.exp(s - m)) + (N - C) * jnp.exp(-m)
    wgt = jnp.exp(s - m) / denom
    live[...] = jnp.maximum(acc * wgt, 0.0)
    live_copy = pltpu.make_async_copy(
        live, out_ref.at[pl.ds(0, C), :], insems.at[5])
    live_copy.start()
    copies.append(live_copy)
    for cpy in copies:
        cpy.wait()


def kernel(x, edge_type, W, nodevec1, nodevec2, rel_table, attention):
    del nodevec1, nodevec2
    t = edge_type.reshape(C, C)
    rel = rel_table.reshape(NUM_REL * C, C)
    att = jnp.broadcast_to(attention.reshape(1, C), (8, C))
    any_spec = pl.BlockSpec(memory_space=pl.ANY)
    return pl.pallas_call(
        _kgat_kernel,
        grid=(1,),
        in_specs=[any_spec] * 5,
        out_specs=any_spec,
        out_shape=jax.ShapeDtypeStruct((N, C), jnp.float32),
        scratch_shapes=[
            pltpu.VMEM((ZROWS, C), jnp.float32),
            pltpu.VMEM((ZROWS, C), jnp.float32),
            pltpu.VMEM((ZROWS, C), jnp.float32),
            pltpu.VMEM((ZROWS, C), jnp.float32),
            pltpu.VMEM((C, C), jnp.float32),
            pltpu.VMEM((C, C), jnp.float32),
            pltpu.VMEM((C, C), jnp.int32),
            pltpu.VMEM((C, C), jnp.float32),
            pltpu.VMEM((NUM_REL * C, C), jnp.float32),
            pltpu.VMEM((8, C), jnp.float32),
            pltpu.SemaphoreType.DMA((NZ,)),
            pltpu.SemaphoreType.DMA((6,)),
        ],
    )(x, t, W, rel, att)
